# Initial kernel scaffold; baseline (speedup 1.0000x reference)
import jax
import jax.numpy as jnp
from jax.experimental import pallas as pl


def kernel(pc1, feature1, W00, b00, W01, b01, W02, b02, W10, b10, W11, b11, W12, b12, W20, b20, W21, b21, W22, b22, Wr, br):
    raise NotImplementedError("write your pallas kernel here")


# speedup vs baseline: 1.3185x; 1.3185x over previous
import jax
import jax.numpy as jnp
from jax.experimental import pallas as pl
from jax.experimental.pallas import tpu as pltpu

_N = 4096
_TQ = 256
_W3 = 0.33
_NEGA0 = -1.0 / (2.0 * 0.02 * 0.02)
_NEGA1 = -1.0 / (2.0 * 0.08 * 0.08)
_NEGA2 = -1.0 / (2.0 * 0.32 * 0.32)


def _net_body(pcq, pct, fea,
              w00, b00, w01, b01, w02, b02,
              w10, b10, w11, b11, w12, b12,
              w20, b20, w21, b21, w22, b22,
              wr, br, out,
              fA, fB, fSB, q2s, s2s, pcqb, pctb):
    q2s[...] = jnp.sum(pcq[...] * pcq[...], axis=1, keepdims=True)
    s2s[...] = jnp.sum(pct[...] * pct[...], axis=0, keepdims=True)
    pcqb[...] = pcq[...].astype(jnp.bfloat16)
    pctb[...] = (pct[...] + pct[...]).astype(jnp.bfloat16)

    def cc(src, w, b, dst_ref, relu=True):
        cout = w.shape[1]
        y = jnp.dot(src.astype(jnp.bfloat16), w[...].astype(jnp.bfloat16),
                    preferred_element_type=jnp.float32) + b[...]
        if relu:
            y = jnp.maximum(y, 0.0)
        dst_ref[:, :cout] = y

    def spatial(nega, cin, src_ref, dst_ref):
        fSB[:, :cin] = src_ref[:, :cin].astype(jnp.bfloat16)

        def tile(i, c):
            r0 = i * _TQ
            qd = jnp.dot(pcqb[pl.ds(r0, _TQ), :], pctb[...],
                         preferred_element_type=jnp.float32)
            d2 = jnp.maximum((q2s[pl.ds(r0, _TQ), :] + s2s[...]) - qd, 0.0)
            e = jnp.exp(d2 * nega)
            e2 = e * e
            e4 = e2 * e2
            e8 = e4 * e4
            e16 = e8 * e8
            k = (e + e4) + e16
            rs = jnp.sum(k, axis=1, keepdims=True)
            num = jnp.dot(k.astype(jnp.bfloat16), fSB[:, :cin],
                          preferred_element_type=jnp.float32)
            dst_ref[pl.ds(r0, _TQ), :cin] = (_W3 * num) / (_W3 * rs + 1e-8)
            return c

        jax.lax.fori_loop(0, _N // _TQ, tile, 0)

    cc(fea[...], w00, b00, fA)
    cc(fA[:, :8], w01, b01, fB)
    spatial(_NEGA0, 16, fB, fA)
    cc(fA[:, :16], w02, b02, fB)
    spatial(_NEGA1, 16, fB, fA)
    cc(fA[:, :16], w10, b10, fB)
    cc(fB[:, :32], w11, b11, fA)
    spatial(_NEGA1, 32, fA, fB)
    cc(fB[:, :32], w12, b12, fA)
    spatial(_NEGA2, 32, fA, fB)
    cc(fB[:, :32], w20, b20, fA)
    cc(fA[:, :64], w21, b21, fB)
    spatial(_NEGA2, 64, fB, fA)
    cc(fA[:, :64], w22, b22, fB)
    y = jnp.dot(fB[:, :64].astype(jnp.bfloat16), wr[...].astype(jnp.bfloat16),
                preferred_element_type=jnp.float32) + br[...]
    out[...] = y


def kernel(pc1, feature1, W00, b00, W01, b01, W02, b02, W10, b10, W11, b11,
           W12, b12, W20, b20, W21, b21, W22, b22, Wr, br):
    pc = pc1[0]
    pcq = jnp.pad(pc, ((0, 0), (0, 5)))
    pct = pcq.T
    fea = feature1[0]
    wts = []
    for w, b in ((W00, b00), (W01, b01), (W02, b02), (W10, b10), (W11, b11),
                 (W12, b12), (W20, b20), (W21, b21), (W22, b22), (Wr, br)):
        wts.append(w.T)
        wts.append(b[None, :])
    out = pl.pallas_call(
        _net_body,
        out_shape=jax.ShapeDtypeStruct((_N, 32), jnp.float32),
        scratch_shapes=[
            pltpu.VMEM((_N, 64), jnp.float32),
            pltpu.VMEM((_N, 64), jnp.float32),
            pltpu.VMEM((_N, 64), jnp.bfloat16),
            pltpu.VMEM((_N, 1), jnp.float32),
            pltpu.VMEM((1, _N), jnp.float32),
            pltpu.VMEM((_N, 8), jnp.bfloat16),
            pltpu.VMEM((8, _N), jnp.bfloat16),
        ],
    )(pcq, pct, fea, *wts)
    return out[None]


# speedup vs baseline: 1.6714x; 1.2676x over previous
import jax
import jax.numpy as jnp
from jax.experimental import pallas as pl
from jax.experimental.pallas import tpu as pltpu

_N = 4096
_TQ = 256
_W3 = 0.33
_LOG2E = 1.4426950408889634
_C0 = -_LOG2E / (2.0 * 0.02 * 0.02)
_C1 = -_LOG2E / (2.0 * 0.08 * 0.08)
_C2 = -_LOG2E / (2.0 * 0.32 * 0.32)


def _net_body(uq, vs, fea,
              w00, b00, w01, b01, w02, b02,
              w10, b10, w11, b11, w12, b12,
              w20, b20, w21, b21, w22, b22,
              wr, br, out,
              fA, fB, fSB):
    def cc(src, w, b, dst_ref, relu=True):
        cout = w.shape[1]
        y = jnp.dot(src.astype(jnp.bfloat16), w[...],
                    preferred_element_type=jnp.float32) + b[...]
        if relu:
            y = jnp.maximum(y, 0.0)
        dst_ref[:, :cout] = y

    def spatial(coef, cin, src_ref, dst_ref):
        fSB[:, :cin] = src_ref[:, :cin].astype(jnp.bfloat16)
        fSB[:, cin:cin + 1] = jnp.ones((_N, 1), jnp.bfloat16)

        def tile(i, c):
            r0 = i * _TQ
            qd = jnp.dot(uq[pl.ds(r0, _TQ), :], vs[...],
                         preferred_element_type=jnp.float32)
            t = jnp.minimum(qd * coef, 0.0)
            e = jnp.exp2(t)
            e2 = e * e
            e4 = e2 * e2
            e8 = e4 * e4
            e16 = e8 * e8
            k = (e + e4) + e16
            num = jnp.dot(k.astype(jnp.bfloat16), fSB[:, :cin + 1],
                          preferred_element_type=jnp.float32)
            rs = num[:, cin:cin + 1]
            dst_ref[pl.ds(r0, _TQ), :cin] = (
                (_W3 * num[:, :cin]) / (_W3 * rs + 1e-8))
            return c

        jax.lax.fori_loop(0, _N // _TQ, tile, 0)

    cc(fea[...], w00, b00, fA)
    cc(fA[:, :8], w01, b01, fB)
    spatial(_C0, 16, fB, fA)
    cc(fA[:, :16], w02, b02, fB)
    spatial(_C1, 16, fB, fA)
    cc(fA[:, :16], w10, b10, fB)
    cc(fB[:, :32], w11, b11, fA)
    spatial(_C1, 32, fA, fB)
    cc(fB[:, :32], w12, b12, fA)
    spatial(_C2, 32, fA, fB)
    cc(fB[:, :32], w20, b20, fA)
    cc(fA[:, :64], w21, b21, fB)
    spatial(_C2, 64, fB, fA)
    cc(fA[:, :64], w22, b22, fB)
    y = jnp.dot(fB[:, :64].astype(jnp.bfloat16), wr[...],
                preferred_element_type=jnp.float32) + br[...]
    out[...] = y


def kernel(pc1, feature1, W00, b00, W01, b01, W02, b02, W10, b10, W11, b11,
           W12, b12, W20, b20, W21, b21, W22, b22, Wr, br):
    pc = pc1[0]
    fea = feature1[0]
    nrm2 = jnp.sum(pc * pc, axis=1, keepdims=True)
    hi = nrm2.astype(jnp.bfloat16).astype(jnp.float32)
    lo = nrm2 - hi
    ones = jnp.ones((_N, 1), jnp.float32)
    zero = jnp.zeros((_N, 1), jnp.float32)
    uq = jnp.concatenate([pc, ones, ones, hi, lo, zero],
                         axis=1).astype(jnp.bfloat16)
    vs = jnp.concatenate([-2.0 * pc, hi, lo, ones, ones, zero],
                         axis=1).astype(jnp.bfloat16).T
    wts = []
    for w, b in ((W00, b00), (W01, b01), (W02, b02), (W10, b10), (W11, b11),
                 (W12, b12), (W20, b20), (W21, b21), (W22, b22), (Wr, br)):
        wts.append(w.T.astype(jnp.bfloat16))
        wts.append(b[None, :])
    out = pl.pallas_call(
        _net_body,
        out_shape=jax.ShapeDtypeStruct((_N, 32), jnp.float32),
        scratch_shapes=[
            pltpu.VMEM((_N, 64), jnp.float32),
            pltpu.VMEM((_N, 64), jnp.float32),
            pltpu.VMEM((_N, 72), jnp.bfloat16),
        ],
    )(uq, vs, fea, *wts)
    return out[None]
